# baseline (speedup 1.0000x reference)
import jax
import jax.numpy as jnp
from jax.experimental import pallas as pl

_K = 25
_KS = 5
_TILE = 256


def _body(coord_ref, mask_ref, x_ref, w_ref, bias_ref, out_ref):
    x = x_ref[...]
    xb = x.astype(jnp.bfloat16)
    z = jnp.dot(xb, w_ref[...], preferred_element_type=jnp.float32)
    F = x.shape[1]

    c = coord_ref[...]
    v = c * jnp.float32(_KS - 1)
    bot = jnp.floor(v)
    frac = v - bot
    boti = bot.astype(jnp.int32)
    f0 = frac[:, 0:1]
    f1 = frac[:, 1:2]
    b0 = boti[:, 0:1]
    b1 = boti[:, 1:2]

    wis = []
    bas = []
    for s in range(4):
        k0 = s % 2
        k1 = s // 2
        wi = jnp.mod(b0 + k0, _KS) + _KS * jnp.mod(b1 + k1, _KS)
        bs = (f0 if k0 else 1.0 - f0) * (f1 if k1 else 1.0 - f1)
        nan = jnp.isnan(bs)
        wis.append(jnp.where(nan, 0, wi))
        bas.append(jnp.where(nan, 0.0, bs))

    acc = z[:, :F]
    for k in range(_K):
        sk = jnp.zeros_like(f0)
        for s in range(4):
            sk = sk + jnp.where(wis[s] == k, bas[s], 0.0)
        acc = acc + sk * z[:, (k + 1) * F:(k + 2) * F]

    out_ref[...] = (acc + bias_ref[...]) * mask_ref[...]


def kernel(x, A, mask, extra, coord, weight, root, bias):
    Bq, Nq, F = x.shape
    E = Bq * Nq
    Ep = ((E + _TILE - 1) // _TILE) * _TILE
    pad = Ep - E

    x2 = jnp.pad(x.reshape(E, F), ((0, pad), (0, 0)))
    coord2 = jnp.pad(coord.reshape(E, 2), ((0, pad), (0, 0)))
    mask2 = jnp.pad(mask.reshape(E, 1), ((0, pad), (0, 0)))
    wflat = jnp.concatenate(
        [root.astype(jnp.bfloat16)[None],
         weight.astype(jnp.bfloat16)], axis=0)
    wflat = wflat.transpose(1, 0, 2).reshape(F, (_K + 1) * F)

    out = pl.pallas_call(
        _body,
        grid=(Ep // _TILE,),
        in_specs=[
            pl.BlockSpec((_TILE, 2), lambda i: (i, 0)),
            pl.BlockSpec((_TILE, 1), lambda i: (i, 0)),
            pl.BlockSpec((_TILE, F), lambda i: (i, 0)),
            pl.BlockSpec((F, (_K + 1) * F), lambda i: (0, 0)),
            pl.BlockSpec((1, F), lambda i: (0, 0)),
        ],
        out_specs=pl.BlockSpec((_TILE, F), lambda i: (i, 0)),
        out_shape=jax.ShapeDtypeStruct((Ep, F), jnp.float32),
    )(coord2, mask2, x2, wflat, bias.reshape(1, F))

    return out[:E].reshape(Bq, Nq, F)


# speedup vs baseline: 1.1120x; 1.1120x over previous
import functools
import jax
import jax.numpy as jnp
from jax import lax
from jax.experimental import pallas as pl
from jax.experimental.pallas import tpu as pltpu
from jax.experimental.pallas import tpu_sc as plsc

_K = 25
_KS = 5
_NC, _NS = 2, 16
_NW = _NC * _NS
_SC_CH = 80
_TILE = 256
_NCELL = 16


def _mesh():
    return plsc.VectorSubcoreMesh(
        core_axis_name="c", subcore_axis_name="s",
        num_cores=_NC, num_subcores=_NS)


def _wid():
    return lax.axis_index("s") * _NC + lax.axis_index("c")


def _cells(c0, c1, gid, n_valid):
    b0 = jnp.minimum(jnp.maximum((c0 * 4.0).astype(jnp.int32), 0), _KS - 2)
    b1 = jnp.minimum(jnp.maximum((c1 * 4.0).astype(jnp.int32), 0), _KS - 2)
    cell = b0 * 4 + b1
    return jnp.where(gid < n_valid, cell, _NCELL)


def _make_sc_hist(E, Ep, NP):
    VC = NP // 16

    @functools.partial(
        pl.kernel, mesh=_mesh(),
        compiler_params=pltpu.CompilerParams(needs_layout_passes=False),
        out_type=jax.ShapeDtypeStruct((_NW * 16,), jnp.int32),
        scratch_types=[
            pltpu.VMEM((NP,), jnp.float32),
            pltpu.VMEM((NP,), jnp.float32),
            pltpu.VMEM((16,), jnp.int32),
        ],
    )
    def k(c0_hbm, c1_hbm, hist_hbm, c0_v, c1_v, cnt_v):
        w = _wid()
        base = w * NP
        pltpu.sync_copy(c0_hbm.at[pl.ds(base, NP)], c0_v)
        pltpu.sync_copy(c1_hbm.at[pl.ds(base, NP)], c1_v)
        lanes = lax.iota(jnp.int32, 16)
        cnt = jnp.zeros((16,), jnp.int32)
        for i in range(VC):
            o = i * 16
            cell = _cells(c0_v[pl.ds(o, 16)], c1_v[pl.ds(o, 16)],
                          base + o + lanes, E)

            def bin_body(b, cnt):
                nb = jnp.sum((cell == b).astype(jnp.int32))
                return cnt + (lanes == b).astype(jnp.int32) * nb

            cnt = lax.fori_loop(0, _NCELL, bin_body, cnt)
        cnt_v[...] = cnt
        pltpu.sync_copy(cnt_v, hist_hbm.at[pl.ds(w * 16, 16)])

    return k


def _make_sc_sort(E, Ep, NP, F, NT, NTmap, EpS):
    VC = NP // 16
    NCH = NP // _SC_CH

    @functools.partial(
        pl.kernel, mesh=_mesh(),
        compiler_params=pltpu.CompilerParams(needs_layout_passes=False),
        out_type=(
            jax.ShapeDtypeStruct((EpS + _TILE, F), jnp.float32),
            jax.ShapeDtypeStruct((EpS + _TILE, 128), jnp.float32),
            jax.ShapeDtypeStruct((_NW * NCH, _SC_CH), jnp.int32),
            jax.ShapeDtypeStruct((NTmap,), jnp.int32),
        ),
        scratch_types=[
            pltpu.VMEM((NP,), jnp.float32),
            pltpu.VMEM((NP,), jnp.float32),
            pltpu.VMEM((_NW * 16,), jnp.int32),
            pltpu.VMEM((NCH, _SC_CH), jnp.int32),
            pltpu.VMEM((NTmap,), jnp.int32),
            pltpu.VMEM((2, _SC_CH, F), jnp.float32),
            pltpu.VMEM((2, _SC_CH, 128), jnp.float32),
            pltpu.SemaphoreType.DMA,
            pltpu.SemaphoreType.DMA,
            pltpu.SemaphoreType.DMA,
            pltpu.SemaphoreType.DMA,
        ],
    )
    def k(c0_hbm, c1_hbm, hist_hbm, x_hbm, cm_hbm,
          xs_hbm, cms_hbm, inv_hbm, tmap_hbm,
          c0_v, c1_v, hist_v, pos_v, tmap_v, xbuf, cmbuf,
          sx0, sx1, sc0, sc1):
        w = _wid()
        base = w * NP
        pltpu.sync_copy(c0_hbm.at[pl.ds(base, NP)], c0_v)
        pltpu.sync_copy(c1_hbm.at[pl.ds(base, NP)], c1_v)
        pltpu.sync_copy(hist_hbm, hist_v)
        lanes = lax.iota(jnp.int32, 16)

        def acc_tot(j, t):
            return t + hist_v[pl.ds(j * 16, 16)]
        totals = lax.fori_loop(0, _NW, acc_tot, jnp.zeros((16,), jnp.int32))
        tceil = ((totals + (_TILE - 1)) // _TILE) * _TILE
        off = plsc.cumsum(tceil) - tceil

        @pl.when(w == 0)
        def _():
            starts = off // _TILE
            ends = (off + tceil) // _TILE
            for j in range(NTmap // 16):
                tid = j * 16 + lanes
                acc = jnp.zeros((16,), jnp.int32) - 1
                for c in range(_NCELL):
                    s = jnp.sum(jnp.where(lanes == c, starts, 0))
                    e = jnp.sum(jnp.where(lanes == c, ends, 0))
                    acc = jnp.where((tid >= s) & (tid < e), c, acc)
                tmap_v[pl.ds(j * 16, 16)] = acc
            pltpu.sync_copy(tmap_v, tmap_hbm)

        def acc_base(j, b):
            row = hist_v[pl.ds(j * 16, 16)]
            return b + (j < w).astype(jnp.int32) * row
        bases = off + lax.fori_loop(0, _NW, acc_base,
                                    jnp.zeros((16,), jnp.int32))

        for i in range(VC):
            o = i * 16
            gid = base + o + lanes
            cell = _cells(c0_v[pl.ds(o, 16)], c1_v[pl.ds(o, 16)], gid, E)

            def bin_body(b, carry):
                pos, bases = carry
                m = cell == b
                r = plsc.cumsum(m.astype(jnp.int32))
                tot = jnp.sum(m.astype(jnp.int32))
                bb = jnp.sum(jnp.where(lanes == b, bases, 0))
                pos = jnp.where(m, bb + r - 1, pos)
                bases = bases + (lanes == b).astype(jnp.int32) * tot
                return pos, bases

            pos, bases = lax.fori_loop(0, _NCELL, bin_body,
                                       (jnp.zeros((16,), jnp.int32), bases))
            pos = jnp.where(cell == _NCELL, EpS + gid - E, pos)
            pos_v[o // _SC_CH, pl.ds(o % _SC_CH, 16)] = pos

        pltpu.sync_copy(pos_v, inv_hbm.at[pl.ds(w * NCH, NCH)])
        hx = [None, None]
        hc = [None, None]
        sems_x = [sx0, sx1]
        sems_c = [sc0, sc1]
        for c in range(NCH):
            s = c % 2
            if hx[s] is not None:
                hx[s].wait()
                hc[s].wait()
            r0 = base + c * _SC_CH
            pltpu.sync_copy(x_hbm.at[pl.ds(r0, _SC_CH)], xbuf.at[s])
            hx[s] = pltpu.async_copy(xbuf.at[s], xs_hbm.at[pos_v.at[c]],
                                     sems_x[s])
            pltpu.sync_copy(cm_hbm.at[pl.ds(r0, _SC_CH)], cmbuf.at[s])
            hc[s] = pltpu.async_copy(cmbuf.at[s], cms_hbm.at[pos_v.at[c]],
                                     sems_c[s])
        for s in range(2):
            if hx[s] is not None:
                hx[s].wait()
                hc[s].wait()

    return k


def _make_sc_unsort(E, Ep, NP, F, EpS):
    NCH = NP // _SC_CH

    @functools.partial(
        pl.kernel, mesh=_mesh(),
        compiler_params=pltpu.CompilerParams(needs_layout_passes=False),
        out_type=jax.ShapeDtypeStruct((Ep, F), jnp.float32),
        scratch_types=[
            pltpu.VMEM((NCH, _SC_CH), jnp.int32),
            pltpu.VMEM((2, _SC_CH, F), jnp.float32),
            pltpu.SemaphoreType.DMA,
            pltpu.SemaphoreType.DMA,
            pltpu.SemaphoreType.DMA,
            pltpu.SemaphoreType.DMA,
        ],
    )
    def k(inv_hbm, os_hbm, out_hbm, idx_v, rowbuf, sg0, sg1, sw0, sw1):
        w = _wid()
        pltpu.sync_copy(inv_hbm.at[pl.ds(w * NCH, NCH)], idx_v)
        sems_g = [sg0, sg1]
        sems_w = [sw0, sw1]
        hw = [None, None]
        for c in range(NCH):
            s = c % 2
            if hw[s] is not None:
                hw[s].wait()
            pltpu.async_copy(os_hbm.at[idx_v.at[c]], rowbuf.at[s],
                             sems_g[s]).wait()
            hw[s] = pltpu.async_copy(
                rowbuf.at[s],
                out_hbm.at[pl.ds(w * NP + c * _SC_CH, _SC_CH)], sems_w[s])
        for s in range(2):
            if hw[s] is not None:
                hw[s].wait()

    return k


def _tc_body(tmap_ref, cms_ref, x_ref, w_ref, root_ref, bias_ref, out_ref):
    t = pl.program_id(0)
    g = tmap_ref[t]

    @pl.when(g >= 0)
    def _():
        x = x_ref[...]
        xb = x.astype(jnp.bfloat16)
        acc = jnp.dot(xb, root_ref[...], preferred_element_type=jnp.float32)

        cm = cms_ref[...]
        v0 = cm[:, 0:1] * jnp.float32(_KS - 1)
        v1 = cm[:, 1:2] * jnp.float32(_KS - 1)
        msk = cm[:, 2:3]
        b0 = jnp.clip(jnp.floor(v0), 0.0, float(_KS - 2))
        b1 = jnp.clip(jnp.floor(v1), 0.0, float(_KS - 2))
        f0 = v0 - b0
        f1 = v1 - b1
        x2 = jnp.concatenate(
            [(x * (1.0 - f0)).astype(jnp.bfloat16),
             (x * f0).astype(jnp.bfloat16)], axis=1)
        p = jnp.dot(x2, w_ref[0], preferred_element_type=jnp.float32)
        F = x.shape[1]
        out_ref[...] = (acc + (1.0 - f1) * p[:, :F] + f1 * p[:, F:]
                        + bias_ref[...]) * msk


def kernel(x, A, mask, extra, coord, weight, root, bias):
    Bq, Nq, F = x.shape
    E = Bq * Nq
    NP = -(-E // (_NW * _SC_CH)) * _SC_CH
    Ep = NP * _NW
    NT = -(-E // _TILE) + _NCELL
    NTmap = -(-NT // 16) * 16
    EpS = NT * _TILE
    assert Ep - E <= _TILE

    x2 = jnp.pad(x.reshape(E, F), ((0, Ep - E), (0, 0)))
    coord2 = coord.reshape(E, 2)
    c0 = jnp.pad(coord2[:, 0], (0, Ep - E))
    c1 = jnp.pad(coord2[:, 1], (0, Ep - E))
    cm = jnp.pad(
        jnp.concatenate([coord2, mask.reshape(E, 1)], axis=1),
        ((0, Ep - E), (0, 125)))

    hist = _make_sc_hist(E, Ep, NP)(c0, c1)
    xs, cms, inv, tmap = _make_sc_sort(E, Ep, NP, F, NT, NTmap, EpS)(
        c0, c1, hist, x2, cm)

    idx = [[[(b0 + r) + _KS * (b1 + s) for s in (0, 1)] for r in (0, 1)]
           for b0 in range(4) for b1 in range(4)]
    wcells = weight[jnp.asarray(idx)]
    wcells = wcells.transpose(0, 1, 3, 2, 4).reshape(_NCELL, 2 * F, 2 * F)
    wcells = wcells.astype(jnp.bfloat16)
    rb = root.astype(jnp.bfloat16)

    out_sorted = pl.pallas_call(
        _tc_body,
        grid_spec=pltpu.PrefetchScalarGridSpec(
            num_scalar_prefetch=1,
            grid=(NT,),
            in_specs=[
                pl.BlockSpec((_TILE, 128), lambda i, tm: (i, 0)),
                pl.BlockSpec((_TILE, F), lambda i, tm: (i, 0)),
                pl.BlockSpec((1, 2 * F, 2 * F),
                             lambda i, tm: (jnp.maximum(tm[i], 0), 0, 0)),
                pl.BlockSpec((F, F), lambda i, tm: (0, 0)),
                pl.BlockSpec((1, F), lambda i, tm: (0, 0)),
            ],
            out_specs=pl.BlockSpec((_TILE, F), lambda i, tm: (i, 0)),
        ),
        out_shape=jax.ShapeDtypeStruct((EpS + _TILE, F), jnp.float32),
    )(tmap, cms, xs, wcells, rb, bias.reshape(1, F))

    out = _make_sc_unsort(E, Ep, NP, F, EpS)(inv, out_sorted)
    return out[:E].reshape(Bq, Nq, F)


# speedup vs baseline: 1.1139x; 1.0017x over previous
import functools
import jax
import jax.numpy as jnp
from jax import lax
from jax.experimental import pallas as pl
from jax.experimental.pallas import tpu as pltpu
from jax.experimental.pallas import tpu_sc as plsc

_K = 25
_KS = 5
_NC, _NS = 2, 16
_NW = _NC * _NS
_SC_CH = 80
_TILE = 256
_NCELL = 16


def _mesh():
    return plsc.VectorSubcoreMesh(
        core_axis_name="c", subcore_axis_name="s",
        num_cores=_NC, num_subcores=_NS)


def _wid():
    return lax.axis_index("s") * _NC + lax.axis_index("c")


def _cells(c0, c1, gid, n_valid):
    b0 = jnp.minimum(jnp.maximum((c0 * 4.0).astype(jnp.int32), 0), _KS - 2)
    b1 = jnp.minimum(jnp.maximum((c1 * 4.0).astype(jnp.int32), 0), _KS - 2)
    cell = b0 * 4 + b1
    return jnp.where(gid < n_valid, cell, _NCELL)


def _make_sc_hist(E, Ep, NP):
    VC = NP // 16

    @functools.partial(
        pl.kernel, mesh=_mesh(),
        compiler_params=pltpu.CompilerParams(needs_layout_passes=False),
        out_type=jax.ShapeDtypeStruct((_NW * 16,), jnp.int32),
        scratch_types=[
            pltpu.VMEM((NP,), jnp.float32),
            pltpu.VMEM((NP,), jnp.float32),
            pltpu.VMEM((16,), jnp.int32),
        ],
    )
    def k(c0_hbm, c1_hbm, hist_hbm, c0_v, c1_v, cnt_v):
        w = _wid()
        base = w * NP
        pltpu.sync_copy(c0_hbm.at[pl.ds(base, NP)], c0_v)
        pltpu.sync_copy(c1_hbm.at[pl.ds(base, NP)], c1_v)
        lanes = lax.iota(jnp.int32, 16)
        cnt = jnp.zeros((16,), jnp.int32)
        for i in range(VC):
            o = i * 16
            cell = _cells(c0_v[pl.ds(o, 16)], c1_v[pl.ds(o, 16)],
                          base + o + lanes, E)

            def bin_body(b, cnt):
                nb = jnp.sum((cell == b).astype(jnp.int32))
                return cnt + (lanes == b).astype(jnp.int32) * nb

            cnt = lax.fori_loop(0, _NCELL, bin_body, cnt)
        cnt_v[...] = cnt
        pltpu.sync_copy(cnt_v, hist_hbm.at[pl.ds(w * 16, 16)])

    return k


def _make_sc_sort(E, Ep, NP, F, NT, NTmap, EpS):
    VC = NP // 16
    NCH = NP // _SC_CH

    @functools.partial(
        pl.kernel, mesh=_mesh(),
        compiler_params=pltpu.CompilerParams(needs_layout_passes=False),
        out_type=(
            jax.ShapeDtypeStruct((EpS + _TILE, F), jnp.float32),
            jax.ShapeDtypeStruct((EpS + _TILE, 128), jnp.float32),
            jax.ShapeDtypeStruct((_NW * NCH, _SC_CH), jnp.int32),
            jax.ShapeDtypeStruct((NTmap,), jnp.int32),
        ),
        scratch_types=[
            pltpu.VMEM((NP,), jnp.float32),
            pltpu.VMEM((NP,), jnp.float32),
            pltpu.VMEM((_NW * 16,), jnp.int32),
            pltpu.VMEM((NCH, _SC_CH), jnp.int32),
            pltpu.VMEM((NTmap,), jnp.int32),
            pltpu.VMEM((2, _SC_CH, F), jnp.float32),
            pltpu.VMEM((2, _SC_CH, 128), jnp.float32),
            pltpu.SemaphoreType.DMA,
            pltpu.SemaphoreType.DMA,
            pltpu.SemaphoreType.DMA,
            pltpu.SemaphoreType.DMA,
        ],
    )
    def k(c0_hbm, c1_hbm, hist_hbm, x_hbm, cm_hbm,
          xs_hbm, cms_hbm, inv_hbm, tmap_hbm,
          c0_v, c1_v, hist_v, pos_v, tmap_v, xbuf, cmbuf,
          sx0, sx1, sc0, sc1):
        w = _wid()
        base = w * NP
        pltpu.sync_copy(c0_hbm.at[pl.ds(base, NP)], c0_v)
        pltpu.sync_copy(c1_hbm.at[pl.ds(base, NP)], c1_v)
        pltpu.sync_copy(hist_hbm, hist_v)
        lanes = lax.iota(jnp.int32, 16)

        def acc_tot(j, t):
            return t + hist_v[pl.ds(j * 16, 16)]
        totals = lax.fori_loop(0, _NW, acc_tot, jnp.zeros((16,), jnp.int32))
        tceil = ((totals + (_TILE - 1)) // _TILE) * _TILE
        off = plsc.cumsum(tceil) - tceil

        @pl.when(w == 0)
        def _():
            starts = off // _TILE
            ends = (off + tceil) // _TILE
            for j in range(NTmap // 16):
                tid = j * 16 + lanes
                acc = jnp.zeros((16,), jnp.int32) - 1
                for c in range(_NCELL):
                    s = jnp.sum(jnp.where(lanes == c, starts, 0))
                    e = jnp.sum(jnp.where(lanes == c, ends, 0))
                    acc = jnp.where((tid >= s) & (tid < e), c, acc)
                tmap_v[pl.ds(j * 16, 16)] = acc
            pltpu.sync_copy(tmap_v, tmap_hbm)

        def acc_base(j, b):
            row = hist_v[pl.ds(j * 16, 16)]
            return b + (j < w).astype(jnp.int32) * row
        bases = off + lax.fori_loop(0, _NW, acc_base,
                                    jnp.zeros((16,), jnp.int32))

        for i in range(VC):
            o = i * 16
            gid = base + o + lanes
            cell = _cells(c0_v[pl.ds(o, 16)], c1_v[pl.ds(o, 16)], gid, E)

            def bin_body(b, carry):
                pos, bases = carry
                m = cell == b
                r = plsc.cumsum(m.astype(jnp.int32))
                tot = jnp.sum(m.astype(jnp.int32))
                bb = jnp.sum(jnp.where(lanes == b, bases, 0))
                pos = jnp.where(m, bb + r - 1, pos)
                bases = bases + (lanes == b).astype(jnp.int32) * tot
                return pos, bases

            pos, bases = lax.fori_loop(0, _NCELL, bin_body,
                                       (jnp.zeros((16,), jnp.int32), bases))
            pos = jnp.where(cell == _NCELL, EpS + gid - E, pos)
            pos_v[o // _SC_CH, pl.ds(o % _SC_CH, 16)] = pos

        pltpu.sync_copy(pos_v, inv_hbm.at[pl.ds(w * NCH, NCH)])
        hx = [None, None]
        hc = [None, None]
        sems_x = [sx0, sx1]
        sems_c = [sc0, sc1]
        for c in range(NCH):
            s = c % 2
            if hx[s] is not None:
                hx[s].wait()
                hc[s].wait()
            r0 = base + c * _SC_CH
            pltpu.sync_copy(x_hbm.at[pl.ds(r0, _SC_CH)], xbuf.at[s])
            hx[s] = pltpu.async_copy(xbuf.at[s], xs_hbm.at[pos_v.at[c]],
                                     sems_x[s])
            pltpu.sync_copy(cm_hbm.at[pl.ds(r0, _SC_CH)], cmbuf.at[s])
            hc[s] = pltpu.async_copy(cmbuf.at[s], cms_hbm.at[pos_v.at[c]],
                                     sems_c[s])
        for s in range(2):
            if hx[s] is not None:
                hx[s].wait()
                hc[s].wait()

    return k


def _make_sc_unsort(E, Ep, NP, F, EpS):
    NCH = NP // _SC_CH

    @functools.partial(
        pl.kernel, mesh=_mesh(),
        compiler_params=pltpu.CompilerParams(needs_layout_passes=False),
        out_type=jax.ShapeDtypeStruct((Ep, F), jnp.float32),
        scratch_types=[
            pltpu.VMEM((NCH, _SC_CH), jnp.int32),
            pltpu.VMEM((2, _SC_CH, F), jnp.float32),
            pltpu.SemaphoreType.DMA,
            pltpu.SemaphoreType.DMA,
            pltpu.SemaphoreType.DMA,
            pltpu.SemaphoreType.DMA,
        ],
    )
    def k(inv_hbm, os_hbm, out_hbm, idx_v, rowbuf, sg0, sg1, sw0, sw1):
        w = _wid()
        pltpu.sync_copy(inv_hbm.at[pl.ds(w * NCH, NCH)], idx_v)
        sems_g = [sg0, sg1]
        sems_w = [sw0, sw1]
        hw = [None, None]
        for c in range(NCH):
            s = c % 2
            if hw[s] is not None:
                hw[s].wait()
            pltpu.async_copy(os_hbm.at[idx_v.at[c]], rowbuf.at[s],
                             sems_g[s]).wait()
            hw[s] = pltpu.async_copy(
                rowbuf.at[s],
                out_hbm.at[pl.ds(w * NP + c * _SC_CH, _SC_CH)], sems_w[s])
        for s in range(2):
            if hw[s] is not None:
                hw[s].wait()

    return k


def _tc_body(tmap_ref, cms_ref, x_ref, w_ref, root_ref, bias_ref, out_ref):
    t = pl.program_id(0)
    g = tmap_ref[t]

    @pl.when(g >= 0)
    def _():
        x = x_ref[...]
        xb = x.astype(jnp.bfloat16)
        acc = jnp.dot(xb, root_ref[...], preferred_element_type=jnp.float32)

        cm = cms_ref[...]
        v0 = cm[:, 0:1] * jnp.float32(_KS - 1)
        v1 = cm[:, 1:2] * jnp.float32(_KS - 1)
        msk = cm[:, 2:3]
        b0 = jnp.clip(jnp.floor(v0), 0.0, float(_KS - 2))
        b1 = jnp.clip(jnp.floor(v1), 0.0, float(_KS - 2))
        f0 = v0 - b0
        f1 = v1 - b1
        x2 = jnp.concatenate(
            [(x * (1.0 - f0)).astype(jnp.bfloat16),
             (x * f0).astype(jnp.bfloat16)], axis=1)
        p = jnp.dot(x2, w_ref[0], preferred_element_type=jnp.float32)
        F = x.shape[1]
        out_ref[...] = (acc + (1.0 - f1) * p[:, :F] + f1 * p[:, F:]
                        + bias_ref[...]) * msk


def kernel(x, A, mask, extra, coord, weight, root, bias):
    Bq, Nq, F = x.shape
    E = Bq * Nq
    NP = -(-E // (_NW * _SC_CH)) * _SC_CH
    Ep = NP * _NW
    NT = -(-E // _TILE) + _NCELL
    NTmap = -(-NT // 16) * 16
    EpS = NT * _TILE
    assert Ep - E <= _TILE

    x2 = jnp.pad(x.reshape(E, F), ((0, Ep - E), (0, 0)))
    coord2 = coord.reshape(E, 2)
    c0 = jnp.pad(coord2[:, 0], (0, Ep - E))
    c1 = jnp.pad(coord2[:, 1], (0, Ep - E))
    cm = jnp.pad(
        jnp.concatenate([coord2, mask.reshape(E, 1)], axis=1),
        ((0, Ep - E), (0, 125)))

    hist = _make_sc_hist(E, Ep, NP)(c0, c1)
    xs, cms, inv, tmap = _make_sc_sort(E, Ep, NP, F, NT, NTmap, EpS)(
        c0, c1, hist, x2, cm)

    idx = [[[(b0 + r) + _KS * (b1 + s) for s in (0, 1)] for r in (0, 1)]
           for b0 in range(4) for b1 in range(4)]
    wcells = weight[jnp.asarray(idx)]
    wcells = wcells.transpose(0, 1, 3, 2, 4).reshape(_NCELL, 2 * F, 2 * F)
    wcells = wcells.astype(jnp.bfloat16)
    rb = root.astype(jnp.bfloat16)

    out_sorted = pl.pallas_call(
        _tc_body,
        grid_spec=pltpu.PrefetchScalarGridSpec(
            num_scalar_prefetch=1,
            grid=(NT,),
            in_specs=[
                pl.BlockSpec((_TILE, 128), lambda i, tm: (i, 0)),
                pl.BlockSpec((_TILE, F), lambda i, tm: (i, 0)),
                pl.BlockSpec((1, 2 * F, 2 * F),
                             lambda i, tm: (jnp.maximum(tm[i], 0), 0, 0)),
                pl.BlockSpec((F, F), lambda i, tm: (0, 0)),
                pl.BlockSpec((1, F), lambda i, tm: (0, 0)),
            ],
            out_specs=pl.BlockSpec(
                (_TILE, F),
                lambda i, tm: (jnp.where(tm[i] < 0, NT, i), 0)),
        ),
        out_shape=jax.ShapeDtypeStruct((EpS + _TILE, F), jnp.float32),
    )(tmap, cms, xs, wcells, rb, bias.reshape(1, F))

    out = _make_sc_unsort(E, Ep, NP, F, EpS)(inv, out_sorted)
    return out[:E].reshape(Bq, Nq, F)


# speedup vs baseline: 1.1632x; 1.0443x over previous
import functools
import jax
import jax.numpy as jnp
from jax import lax
from jax.experimental import pallas as pl
from jax.experimental.pallas import tpu as pltpu
from jax.experimental.pallas import tpu_sc as plsc

_K = 25
_KS = 5
_NC, _NS = 2, 16
_NW = _NC * _NS
_SC_CH = 80
_TILE = 512
_NCELL = 16


def _mesh():
    return plsc.VectorSubcoreMesh(
        core_axis_name="c", subcore_axis_name="s",
        num_cores=_NC, num_subcores=_NS)


def _wid():
    return lax.axis_index("s") * _NC + lax.axis_index("c")


def _cells(c0, c1, gid, n_valid):
    b0 = jnp.minimum(jnp.maximum((c0 * 4.0).astype(jnp.int32), 0), _KS - 2)
    b1 = jnp.minimum(jnp.maximum((c1 * 4.0).astype(jnp.int32), 0), _KS - 2)
    cell = b0 * 4 + b1
    return jnp.where(gid < n_valid, cell, _NCELL)


def _make_sc_hist(E, Ep, NP):
    VC = NP // 16

    @functools.partial(
        pl.kernel, mesh=_mesh(),
        compiler_params=pltpu.CompilerParams(needs_layout_passes=False),
        out_type=jax.ShapeDtypeStruct((_NW * 16,), jnp.int32),
        scratch_types=[
            pltpu.VMEM((NP,), jnp.float32),
            pltpu.VMEM((NP,), jnp.float32),
            pltpu.VMEM((16,), jnp.int32),
        ],
    )
    def k(c0_hbm, c1_hbm, hist_hbm, c0_v, c1_v, cnt_v):
        w = _wid()
        base = w * NP
        pltpu.sync_copy(c0_hbm.at[pl.ds(base, NP)], c0_v)
        pltpu.sync_copy(c1_hbm.at[pl.ds(base, NP)], c1_v)
        lanes = lax.iota(jnp.int32, 16)
        cnt = jnp.zeros((16,), jnp.int32)
        for i in range(VC):
            o = i * 16
            cell = _cells(c0_v[pl.ds(o, 16)], c1_v[pl.ds(o, 16)],
                          base + o + lanes, E)

            def bin_body(b, cnt):
                nb = jnp.sum((cell == b).astype(jnp.int32))
                return cnt + (lanes == b).astype(jnp.int32) * nb

            cnt = lax.fori_loop(0, _NCELL, bin_body, cnt)
        cnt_v[...] = cnt
        pltpu.sync_copy(cnt_v, hist_hbm.at[pl.ds(w * 16, 16)])

    return k


def _make_sc_sort(E, Ep, NP, F, NT, NTmap, EpS):
    VC = NP // 16
    NCH = NP // _SC_CH

    @functools.partial(
        pl.kernel, mesh=_mesh(),
        compiler_params=pltpu.CompilerParams(needs_layout_passes=False),
        out_type=(
            jax.ShapeDtypeStruct((EpS + _TILE, F), jnp.float32),
            jax.ShapeDtypeStruct((EpS + _TILE, 128), jnp.float32),
            jax.ShapeDtypeStruct((_NW * NCH, _SC_CH), jnp.int32),
            jax.ShapeDtypeStruct((NTmap,), jnp.int32),
        ),
        scratch_types=[
            pltpu.VMEM((NP,), jnp.float32),
            pltpu.VMEM((NP,), jnp.float32),
            pltpu.VMEM((_NW * 16,), jnp.int32),
            pltpu.VMEM((NCH, _SC_CH), jnp.int32),
            pltpu.VMEM((NTmap,), jnp.int32),
            pltpu.VMEM((2, _SC_CH, F), jnp.float32),
            pltpu.VMEM((2, _SC_CH, 128), jnp.float32),
            pltpu.SemaphoreType.DMA,
            pltpu.SemaphoreType.DMA,
            pltpu.SemaphoreType.DMA,
            pltpu.SemaphoreType.DMA,
        ],
    )
    def k(c0_hbm, c1_hbm, hist_hbm, x_hbm, cm_hbm,
          xs_hbm, cms_hbm, inv_hbm, tmap_hbm,
          c0_v, c1_v, hist_v, pos_v, tmap_v, xbuf, cmbuf,
          sx0, sx1, sc0, sc1):
        w = _wid()
        base = w * NP
        pltpu.sync_copy(c0_hbm.at[pl.ds(base, NP)], c0_v)
        pltpu.sync_copy(c1_hbm.at[pl.ds(base, NP)], c1_v)
        pltpu.sync_copy(hist_hbm, hist_v)
        lanes = lax.iota(jnp.int32, 16)

        def acc_tot(j, t):
            return t + hist_v[pl.ds(j * 16, 16)]
        totals = lax.fori_loop(0, _NW, acc_tot, jnp.zeros((16,), jnp.int32))
        tceil = ((totals + (_TILE - 1)) // _TILE) * _TILE
        off = plsc.cumsum(tceil) - tceil

        @pl.when(w == 0)
        def _():
            starts = off // _TILE
            ends = (off + tceil) // _TILE
            for j in range(NTmap // 16):
                tid = j * 16 + lanes
                acc = jnp.zeros((16,), jnp.int32) - 1
                for c in range(_NCELL):
                    s = jnp.sum(jnp.where(lanes == c, starts, 0))
                    e = jnp.sum(jnp.where(lanes == c, ends, 0))
                    acc = jnp.where((tid >= s) & (tid < e), c, acc)
                tmap_v[pl.ds(j * 16, 16)] = acc
            pltpu.sync_copy(tmap_v, tmap_hbm)

        def acc_base(j, b):
            row = hist_v[pl.ds(j * 16, 16)]
            return b + (j < w).astype(jnp.int32) * row
        bases = off + lax.fori_loop(0, _NW, acc_base,
                                    jnp.zeros((16,), jnp.int32))

        for i in range(VC):
            o = i * 16
            gid = base + o + lanes
            cell = _cells(c0_v[pl.ds(o, 16)], c1_v[pl.ds(o, 16)], gid, E)

            def bin_body(b, carry):
                pos, bases = carry
                m = cell == b
                r = plsc.cumsum(m.astype(jnp.int32))
                tot = jnp.sum(m.astype(jnp.int32))
                bb = jnp.sum(jnp.where(lanes == b, bases, 0))
                pos = jnp.where(m, bb + r - 1, pos)
                bases = bases + (lanes == b).astype(jnp.int32) * tot
                return pos, bases

            pos, bases = lax.fori_loop(0, _NCELL, bin_body,
                                       (jnp.zeros((16,), jnp.int32), bases))
            pos = jnp.where(cell == _NCELL, EpS + gid - E, pos)
            pos_v[o // _SC_CH, pl.ds(o % _SC_CH, 16)] = pos

        pltpu.sync_copy(pos_v, inv_hbm.at[pl.ds(w * NCH, NCH)])
        hx = [None, None]
        hc = [None, None]
        sems_x = [sx0, sx1]
        sems_c = [sc0, sc1]
        for c in range(NCH):
            s = c % 2
            if hx[s] is not None:
                hx[s].wait()
                hc[s].wait()
            r0 = base + c * _SC_CH
            pltpu.sync_copy(x_hbm.at[pl.ds(r0, _SC_CH)], xbuf.at[s])
            hx[s] = pltpu.async_copy(xbuf.at[s], xs_hbm.at[pos_v.at[c]],
                                     sems_x[s])
            pltpu.sync_copy(cm_hbm.at[pl.ds(r0, _SC_CH)], cmbuf.at[s])
            hc[s] = pltpu.async_copy(cmbuf.at[s], cms_hbm.at[pos_v.at[c]],
                                     sems_c[s])
        for s in range(2):
            if hx[s] is not None:
                hx[s].wait()
                hc[s].wait()

    return k


def _make_sc_unsort(E, Ep, NP, F, EpS):
    NCH = NP // _SC_CH

    @functools.partial(
        pl.kernel, mesh=_mesh(),
        compiler_params=pltpu.CompilerParams(needs_layout_passes=False),
        out_type=jax.ShapeDtypeStruct((Ep, F), jnp.float32),
        scratch_types=[
            pltpu.VMEM((NCH, _SC_CH), jnp.int32),
            pltpu.VMEM((2, _SC_CH, F), jnp.float32),
            pltpu.SemaphoreType.DMA,
            pltpu.SemaphoreType.DMA,
            pltpu.SemaphoreType.DMA,
            pltpu.SemaphoreType.DMA,
        ],
    )
    def k(inv_hbm, os_hbm, out_hbm, idx_v, rowbuf, sg0, sg1, sw0, sw1):
        w = _wid()
        pltpu.sync_copy(inv_hbm.at[pl.ds(w * NCH, NCH)], idx_v)
        sems_g = [sg0, sg1]
        sems_w = [sw0, sw1]
        hw = [None, None]
        for c in range(NCH):
            s = c % 2
            if hw[s] is not None:
                hw[s].wait()
            pltpu.async_copy(os_hbm.at[idx_v.at[c]], rowbuf.at[s],
                             sems_g[s]).wait()
            hw[s] = pltpu.async_copy(
                rowbuf.at[s],
                out_hbm.at[pl.ds(w * NP + c * _SC_CH, _SC_CH)], sems_w[s])
        for s in range(2):
            if hw[s] is not None:
                hw[s].wait()

    return k


def _tc_body(tmap_ref, cms_ref, x_ref, w_ref, root_ref, bias_ref, out_ref):
    t = pl.program_id(0)
    g = tmap_ref[t]

    @pl.when(g >= 0)
    def _():
        x = x_ref[...]
        xb = x.astype(jnp.bfloat16)
        acc = jnp.dot(xb, root_ref[...], preferred_element_type=jnp.float32)

        cm = cms_ref[...]
        v0 = cm[:, 0:1] * jnp.float32(_KS - 1)
        v1 = cm[:, 1:2] * jnp.float32(_KS - 1)
        msk = cm[:, 2:3]
        b0 = jnp.clip(jnp.floor(v0), 0.0, float(_KS - 2))
        b1 = jnp.clip(jnp.floor(v1), 0.0, float(_KS - 2))
        f0 = v0 - b0
        f1 = v1 - b1
        x2 = jnp.concatenate(
            [(x * (1.0 - f0)).astype(jnp.bfloat16),
             (x * f0).astype(jnp.bfloat16)], axis=1)
        p = jnp.dot(x2, w_ref[0], preferred_element_type=jnp.float32)
        F = x.shape[1]
        out_ref[...] = (acc + (1.0 - f1) * p[:, :F] + f1 * p[:, F:]
                        + bias_ref[...]) * msk


def kernel(x, A, mask, extra, coord, weight, root, bias):
    Bq, Nq, F = x.shape
    E = Bq * Nq
    NP = -(-E // (_NW * _SC_CH)) * _SC_CH
    Ep = NP * _NW
    NT = -(-E // _TILE) + _NCELL
    NTmap = -(-NT // 16) * 16
    EpS = NT * _TILE
    assert Ep - E <= _TILE

    x2 = jnp.pad(x.reshape(E, F), ((0, Ep - E), (0, 0)))
    coord2 = coord.reshape(E, 2)
    c0 = jnp.pad(coord2[:, 0], (0, Ep - E))
    c1 = jnp.pad(coord2[:, 1], (0, Ep - E))
    cm = jnp.pad(
        jnp.concatenate([coord2, mask.reshape(E, 1)], axis=1),
        ((0, Ep - E), (0, 125)))

    hist = _make_sc_hist(E, Ep, NP)(c0, c1)
    xs, cms, inv, tmap = _make_sc_sort(E, Ep, NP, F, NT, NTmap, EpS)(
        c0, c1, hist, x2, cm)

    idx = [[[(b0 + r) + _KS * (b1 + s) for s in (0, 1)] for r in (0, 1)]
           for b0 in range(4) for b1 in range(4)]
    wcells = weight[jnp.asarray(idx)]
    wcells = wcells.transpose(0, 1, 3, 2, 4).reshape(_NCELL, 2 * F, 2 * F)
    wcells = wcells.astype(jnp.bfloat16)
    rb = root.astype(jnp.bfloat16)

    out_sorted = pl.pallas_call(
        _tc_body,
        grid_spec=pltpu.PrefetchScalarGridSpec(
            num_scalar_prefetch=1,
            grid=(NT,),
            in_specs=[
                pl.BlockSpec((_TILE, 128), lambda i, tm: (i, 0)),
                pl.BlockSpec((_TILE, F), lambda i, tm: (i, 0)),
                pl.BlockSpec((1, 2 * F, 2 * F),
                             lambda i, tm: (jnp.maximum(tm[i], 0), 0, 0)),
                pl.BlockSpec((F, F), lambda i, tm: (0, 0)),
                pl.BlockSpec((1, F), lambda i, tm: (0, 0)),
            ],
            out_specs=pl.BlockSpec(
                (_TILE, F),
                lambda i, tm: (jnp.where(tm[i] < 0, NT, i), 0)),
        ),
        out_shape=jax.ShapeDtypeStruct((EpS + _TILE, F), jnp.float32),
    )(tmap, cms, xs, wcells, rb, bias.reshape(1, F))

    out = _make_sc_unsort(E, Ep, NP, F, EpS)(inv, out_sorted)
    return out[:E].reshape(Bq, Nq, F)


# speedup vs baseline: 1.2401x; 1.0661x over previous
import functools
import jax
import jax.numpy as jnp
from jax import lax
from jax.experimental import pallas as pl
from jax.experimental.pallas import tpu as pltpu
from jax.experimental.pallas import tpu_sc as plsc

_K = 25
_KS = 5
_NC, _NS = 2, 16
_NW = _NC * _NS
_SC_CH = 80
_TILE = 256
_NCELL = 16


def _mesh():
    return plsc.VectorSubcoreMesh(
        core_axis_name="c", subcore_axis_name="s",
        num_cores=_NC, num_subcores=_NS)


def _wid():
    return lax.axis_index("s") * _NC + lax.axis_index("c")


def _cells(c0, c1, gid, n_valid):
    b0 = jnp.minimum(jnp.maximum((c0 * 4.0).astype(jnp.int32), 0), _KS - 2)
    b1 = jnp.minimum(jnp.maximum((c1 * 4.0).astype(jnp.int32), 0), _KS - 2)
    cell = b0 * 4 + b1
    return jnp.where(gid < n_valid, cell, _NCELL)


def _make_sc_hist(E, Ep, NP):
    VC = NP // 16

    @functools.partial(
        pl.kernel, mesh=_mesh(),
        compiler_params=pltpu.CompilerParams(needs_layout_passes=False),
        out_type=jax.ShapeDtypeStruct((_NW * 16,), jnp.int32),
        scratch_types=[
            pltpu.VMEM((NP,), jnp.float32),
            pltpu.VMEM((NP,), jnp.float32),
            pltpu.VMEM((16,), jnp.int32),
        ],
    )
    def k(c0_hbm, c1_hbm, hist_hbm, c0_v, c1_v, cnt_v):
        w = _wid()
        base = w * NP
        pltpu.sync_copy(c0_hbm.at[pl.ds(base, NP)], c0_v)
        pltpu.sync_copy(c1_hbm.at[pl.ds(base, NP)], c1_v)
        lanes = lax.iota(jnp.int32, 16)
        cnt = jnp.zeros((16,), jnp.int32)
        for i in range(VC):
            o = i * 16
            cell = _cells(c0_v[pl.ds(o, 16)], c1_v[pl.ds(o, 16)],
                          base + o + lanes, E)

            def bin_body(b, cnt):
                nb = jnp.sum((cell == b).astype(jnp.int32))
                return cnt + (lanes == b).astype(jnp.int32) * nb

            cnt = lax.fori_loop(0, _NCELL, bin_body, cnt)
        cnt_v[...] = cnt
        pltpu.sync_copy(cnt_v, hist_hbm.at[pl.ds(w * 16, 16)])

    return k


def _make_sc_sort(E, Ep, NP, F):
    VC = NP // 16
    NCH = NP // _SC_CH

    @functools.partial(
        pl.kernel, mesh=_mesh(),
        compiler_params=pltpu.CompilerParams(needs_layout_passes=False),
        out_type=(
            jax.ShapeDtypeStruct((Ep, F), jnp.float32),
            jax.ShapeDtypeStruct((Ep, 128), jnp.float32),
            jax.ShapeDtypeStruct((_NW * NCH, _SC_CH), jnp.int32),
            jax.ShapeDtypeStruct((32,), jnp.int32),
        ),
        scratch_types=[
            pltpu.VMEM((NP,), jnp.float32),
            pltpu.VMEM((NP,), jnp.float32),
            pltpu.VMEM((_NW * 16,), jnp.int32),
            pltpu.VMEM((NCH, _SC_CH), jnp.int32),
            pltpu.VMEM((32,), jnp.int32),
            pltpu.VMEM((2, _SC_CH, F), jnp.float32),
            pltpu.VMEM((2, _SC_CH, 128), jnp.float32),
            pltpu.SemaphoreType.DMA,
            pltpu.SemaphoreType.DMA,
            pltpu.SemaphoreType.DMA,
            pltpu.SemaphoreType.DMA,
        ],
    )
    def k(c0_hbm, c1_hbm, hist_hbm, x_hbm, cm_hbm,
          xs_hbm, cms_hbm, inv_hbm, offs_hbm,
          c0_v, c1_v, hist_v, pos_v, offs_v, xbuf, cmbuf,
          sx0, sx1, sc0, sc1):
        w = _wid()
        base = w * NP
        pltpu.sync_copy(c0_hbm.at[pl.ds(base, NP)], c0_v)
        pltpu.sync_copy(c1_hbm.at[pl.ds(base, NP)], c1_v)
        pltpu.sync_copy(hist_hbm, hist_v)
        lanes = lax.iota(jnp.int32, 16)

        def acc_tot(j, t):
            return t + hist_v[pl.ds(j * 16, 16)]
        totals = lax.fori_loop(0, _NW, acc_tot, jnp.zeros((16,), jnp.int32))
        off = plsc.cumsum(totals) - totals

        @pl.when(w == 0)
        def _():
            offs_v[pl.ds(0, 16)] = off
            offs_v[pl.ds(16, 16)] = (lanes == 0).astype(jnp.int32) * E
            pltpu.sync_copy(offs_v, offs_hbm)

        def acc_base(j, b):
            row = hist_v[pl.ds(j * 16, 16)]
            return b + (j < w).astype(jnp.int32) * row
        bases = off + lax.fori_loop(0, _NW, acc_base,
                                    jnp.zeros((16,), jnp.int32))

        for i in range(VC):
            o = i * 16
            gid = base + o + lanes
            cell = _cells(c0_v[pl.ds(o, 16)], c1_v[pl.ds(o, 16)], gid, E)

            def bin_body(b, carry):
                pos, bases = carry
                m = cell == b
                r = plsc.cumsum(m.astype(jnp.int32))
                tot = jnp.sum(m.astype(jnp.int32))
                bb = jnp.sum(jnp.where(lanes == b, bases, 0))
                pos = jnp.where(m, bb + r - 1, pos)
                bases = bases + (lanes == b).astype(jnp.int32) * tot
                return pos, bases

            pos, bases = lax.fori_loop(0, _NCELL, bin_body,
                                       (jnp.zeros((16,), jnp.int32), bases))
            pos = jnp.where(cell == _NCELL, gid, pos)
            pos_v[o // _SC_CH, pl.ds(o % _SC_CH, 16)] = pos

        pltpu.sync_copy(pos_v, inv_hbm.at[pl.ds(w * NCH, NCH)])
        hx = [None, None]
        hc = [None, None]
        sems_x = [sx0, sx1]
        sems_c = [sc0, sc1]
        for c in range(NCH):
            s = c % 2
            if hx[s] is not None:
                hx[s].wait()
                hc[s].wait()
            r0 = base + c * _SC_CH
            pltpu.sync_copy(x_hbm.at[pl.ds(r0, _SC_CH)], xbuf.at[s])
            hx[s] = pltpu.async_copy(xbuf.at[s], xs_hbm.at[pos_v.at[c]],
                                     sems_x[s])
            pltpu.sync_copy(cm_hbm.at[pl.ds(r0, _SC_CH)], cmbuf.at[s])
            hc[s] = pltpu.async_copy(cmbuf.at[s], cms_hbm.at[pos_v.at[c]],
                                     sems_c[s])
        for s in range(2):
            if hx[s] is not None:
                hx[s].wait()
                hc[s].wait()

    return k


def _make_sc_unsort(E, Ep, NP, F):
    NCH = NP // _SC_CH

    @functools.partial(
        pl.kernel, mesh=_mesh(),
        compiler_params=pltpu.CompilerParams(needs_layout_passes=False),
        out_type=jax.ShapeDtypeStruct((Ep, F), jnp.float32),
        scratch_types=[
            pltpu.VMEM((NCH, _SC_CH), jnp.int32),
            pltpu.VMEM((2, _SC_CH, F), jnp.float32),
            pltpu.SemaphoreType.DMA,
            pltpu.SemaphoreType.DMA,
            pltpu.SemaphoreType.DMA,
            pltpu.SemaphoreType.DMA,
        ],
    )
    def k(inv_hbm, os_hbm, out_hbm, idx_v, rowbuf, sg0, sg1, sw0, sw1):
        w = _wid()
        pltpu.sync_copy(inv_hbm.at[pl.ds(w * NCH, NCH)], idx_v)
        sems_g = [sg0, sg1]
        sems_w = [sw0, sw1]
        hw = [None, None]
        for c in range(NCH):
            s = c % 2
            if hw[s] is not None:
                hw[s].wait()
            pltpu.async_copy(os_hbm.at[idx_v.at[c]], rowbuf.at[s],
                             sems_g[s]).wait()
            hw[s] = pltpu.async_copy(
                rowbuf.at[s],
                out_hbm.at[pl.ds(w * NP + c * _SC_CH, _SC_CH)], sems_w[s])
        for s in range(2):
            if hw[s] is not None:
                hw[s].wait()

    return k


def _tc_body(offs_ref, cms_ref, x_ref, w_ref, root_ref, bias_ref, out_ref,
             pl_ref, pr_ref):
    t = pl.program_id(0)
    x = x_ref[...]
    xb = x.astype(jnp.bfloat16)
    acc = jnp.dot(xb, root_ref[...], preferred_element_type=jnp.float32)

    cm = cms_ref[...]
    v0 = cm[:, 0:1] * jnp.float32(_KS - 1)
    v1 = cm[:, 1:2] * jnp.float32(_KS - 1)
    msk = cm[:, 2:3]
    b0 = jnp.clip(jnp.floor(v0), 0.0, float(_KS - 2))
    b1 = jnp.clip(jnp.floor(v1), 0.0, float(_KS - 2))
    f0 = v0 - b0
    f1 = v1 - b1
    xlo = (x * (1.0 - f0)).astype(jnp.bfloat16)
    xhi = (x * f0).astype(jnp.bfloat16)

    T = x.shape[0]
    F = x.shape[1]
    rowid = t * T + lax.broadcasted_iota(jnp.int32, (T, 1), 0)
    pl_ref[...] = jnp.zeros_like(pl_ref)
    pr_ref[...] = jnp.zeros_like(pr_ref)

    for g in range(_NCELL):
        gb0, gb1 = g // 4, g % 4
        lo = jnp.maximum(offs_ref[g], t * T)
        hi = jnp.minimum(offs_ref[g + 1], (t + 1) * T)

        @pl.when(hi > lo)
        def _(gb0=gb0, gb1=gb1, lo=lo, hi=hi):
            m = (rowid >= lo) & (rowid < hi)
            xl = jnp.where(m, xlo, jnp.bfloat16(0))
            xh = jnp.where(m, xhi, jnp.bfloat16(0))
            i00 = gb0 + _KS * gb1
            i10 = gb0 + 1 + _KS * gb1
            i01 = gb0 + _KS * (gb1 + 1)
            i11 = gb0 + 1 + _KS * (gb1 + 1)
            pl_ref[...] += (
                jnp.dot(xl, w_ref[i00], preferred_element_type=jnp.float32)
                + jnp.dot(xh, w_ref[i10], preferred_element_type=jnp.float32))
            pr_ref[...] += (
                jnp.dot(xl, w_ref[i01], preferred_element_type=jnp.float32)
                + jnp.dot(xh, w_ref[i11], preferred_element_type=jnp.float32))

    out_ref[...] = (acc + (1.0 - f1) * pl_ref[...] + f1 * pr_ref[...]
                    + bias_ref[...]) * msk


def kernel(x, A, mask, extra, coord, weight, root, bias):
    Bq, Nq, F = x.shape
    E = Bq * Nq
    NP = -(-E // (_NW * _SC_CH)) * _SC_CH
    Ep = NP * _NW
    assert Ep % _TILE == 0

    x2 = jnp.pad(x.reshape(E, F), ((0, Ep - E), (0, 0)))
    coord2 = coord.reshape(E, 2)
    c0 = jnp.pad(coord2[:, 0], (0, Ep - E))
    c1 = jnp.pad(coord2[:, 1], (0, Ep - E))
    cm = jnp.pad(
        jnp.concatenate([coord2, mask.reshape(E, 1)], axis=1),
        ((0, Ep - E), (0, 125)))

    hist = _make_sc_hist(E, Ep, NP)(c0, c1)
    xs, cms, inv, offs = _make_sc_sort(E, Ep, NP, F)(c0, c1, hist, x2, cm)

    wb = weight.astype(jnp.bfloat16)
    rb = root.astype(jnp.bfloat16)

    out_sorted = pl.pallas_call(
        _tc_body,
        grid_spec=pltpu.PrefetchScalarGridSpec(
            num_scalar_prefetch=1,
            grid=(Ep // _TILE,),
            in_specs=[
                pl.BlockSpec((_TILE, 128), lambda i, offs: (i, 0)),
                pl.BlockSpec((_TILE, F), lambda i, offs: (i, 0)),
                pl.BlockSpec((_K, F, F), lambda i, offs: (0, 0, 0)),
                pl.BlockSpec((F, F), lambda i, offs: (0, 0)),
                pl.BlockSpec((1, F), lambda i, offs: (0, 0)),
            ],
            out_specs=pl.BlockSpec((_TILE, F), lambda i, offs: (i, 0)),
            scratch_shapes=[
                pltpu.VMEM((_TILE, F), jnp.float32),
                pltpu.VMEM((_TILE, F), jnp.float32),
            ],
        ),
        out_shape=jax.ShapeDtypeStruct((Ep, F), jnp.float32),
    )(offs, cms, xs, wb, rb, bias.reshape(1, F))

    out = _make_sc_unsort(E, Ep, NP, F)(inv, out_sorted)
    return out[:E].reshape(Bq, Nq, F)
